# Initial kernel scaffold; baseline (speedup 1.0000x reference)
import jax
import jax.numpy as jnp
from jax.experimental import pallas as pl


def kernel(x, edge_index, eps, W1, b1, g1, bt1, W2, b2, g2, bt2):
    raise NotImplementedError("write your pallas kernel here")


# speedup vs baseline: 4.3331x; 4.3331x over previous
import functools

import jax
import jax.numpy as jnp
from jax import lax
from jax.experimental import pallas as pl
from jax.experimental.pallas import tpu as pltpu
from jax.experimental.pallas import tpu_sc as plsc

N_NODES = 10000
D = 128
NC = 2
NS = 16
NW = NC * NS
EB = 128
ACC_ROWS = 10240
ZROWS = ACC_ROWS // NS


def _make_sc_segment_sum(e_pad):
    cpt = e_pad // (NW * EB)
    mesh = plsc.VectorSubcoreMesh(core_axis_name="c", subcore_axis_name="s")

    @functools.partial(
        pl.kernel,
        mesh=mesh,
        out_type=jax.ShapeDtypeStruct((NC, ACC_ROWS, D), jnp.float32),
        scratch_types=[
            pltpu.VMEM((EB,), jnp.int32),
            pltpu.VMEM((EB,), jnp.int32),
            pltpu.VMEM((EB, D), jnp.float32),
            pltpu.VMEM_SHARED((ACC_ROWS, D), jnp.float32),
            pltpu.SemaphoreType.DMA,
        ],
    )
    def seg_sum(src_hbm, dst_hbm, x_hbm, zeros_hbm, out_hbm,
                src_v, dst_v, rows_v, acc, sem):
        c = lax.axis_index("c")
        s = lax.axis_index("s")
        wid = s * NC + c

        pltpu.sync_copy(zeros_hbm, acc.at[pl.ds(s * ZROWS, ZROWS)])
        plsc.subcore_barrier()

        base = wid * cpt * EB

        def body(j, carry):
            off = base + j * EB
            pltpu.sync_copy(src_hbm.at[pl.ds(off, EB)], src_v)
            pltpu.sync_copy(dst_hbm.at[pl.ds(off, EB)], dst_v)
            pltpu.async_copy(x_hbm.at[src_v], rows_v, sem).wait()
            pltpu.sync_copy(rows_v, acc.at[dst_v], add=True)
            return carry

        lax.fori_loop(0, cpt, body, 0)
        plsc.subcore_barrier()

        pltpu.sync_copy(acc.at[pl.ds(s * ZROWS, ZROWS)],
                        out_hbm.at[c, pl.ds(s * ZROWS, ZROWS)])

    return seg_sum


def _mlp_kernel(parts_ref, x_ref, eps_ref,
                w1_ref, b1_ref, g1_ref, bt1_ref,
                w2_ref, b2_ref, g2_ref, bt2_ref, out_ref):
    n = x_ref.shape[0]
    out = (parts_ref[0, :n] + parts_ref[1, :n]
           + (1.0 + eps_ref[0, 0]) * x_ref[...])
    h = jnp.dot(out, w1_ref[...], preferred_element_type=jnp.float32)
    h = h + b1_ref[...]
    mu = jnp.mean(h, axis=0, keepdims=True)
    var = jnp.mean((h - mu) ** 2, axis=0, keepdims=True)
    h = (h - mu) * lax.rsqrt(var + 1e-5) * g1_ref[...] + bt1_ref[...]
    h = jnp.maximum(h, 0.0)
    h = jnp.dot(h, w2_ref[...], preferred_element_type=jnp.float32)
    h = h + b2_ref[...]
    mu = jnp.mean(h, axis=0, keepdims=True)
    var = jnp.mean((h - mu) ** 2, axis=0, keepdims=True)
    h = (h - mu) * lax.rsqrt(var + 1e-5) * g2_ref[...] + bt2_ref[...]
    out_ref[...] = jnp.maximum(h, 0.0)


def kernel(x, edge_index, eps, W1, b1, g1, bt1, W2, b2, g2, bt2):
    n, d = x.shape
    e = edge_index.shape[1]
    cpt = -(-e // (NW * EB))
    e_pad = NW * EB * cpt
    src = edge_index[0]
    dst = edge_index[1]
    pad = e_pad - e
    if pad:
        src = jnp.concatenate([src, jnp.zeros((pad,), jnp.int32)])
        dst = jnp.concatenate([dst, jnp.full((pad,), N_NODES, jnp.int32)])
    zeros = jnp.zeros((ZROWS, d), jnp.float32)

    parts = _make_sc_segment_sum(e_pad)(src, dst, x, zeros)

    out = pl.pallas_call(
        _mlp_kernel,
        out_shape=jax.ShapeDtypeStruct((n, d), jnp.float32),
    )(parts, x, eps.reshape(1, 1),
      W1, b1.reshape(1, d), g1.reshape(1, d), bt1.reshape(1, d),
      W2, b2.reshape(1, d), g2.reshape(1, d), bt2.reshape(1, d))
    return out
